# baseline (speedup 1.0000x reference)
import functools

import jax
import jax.numpy as jnp
from jax.experimental import pallas as pl
from jax.experimental.pallas import tpu as pltpu

E_BLOCK = 12800


def _edge_score_body(ag_ref, bg_ref, m2_ref, out_ref):
    h = jnp.maximum(ag_ref[...] + bg_ref[...], 0.0)
    out_ref[...] = h @ m2_ref[...]


def _edge_score(ag, bg, m2):
    e = ag.shape[0]
    grid = e // E_BLOCK
    return pl.pallas_call(
        _edge_score_body,
        grid=(grid,),
        in_specs=[
            pl.BlockSpec((E_BLOCK, 32), lambda i: (i, 0)),
            pl.BlockSpec((E_BLOCK, 32), lambda i: (i, 0)),
            pl.BlockSpec((32, 1), lambda i: (0, 0)),
        ],
        out_specs=pl.BlockSpec((E_BLOCK, 1), lambda i: (i, 0)),
        out_shape=jax.ShapeDtypeStruct((e, 1), jnp.float32),
    )(ag, bg, m2)


def _seg_mean(x_gathered, seg_ids, n_seg, cnt):
    s = jax.ops.segment_sum(x_gathered, seg_ids, num_segments=n_seg)
    return s / cnt


def kernel(x_account, x_merchant, edge_index, W1l_am, b1l_am, W1r_am,
           W1l_ma, b1l_ma, W1r_ma, W2l_am, b2l_am, W2r_am, W2l_ma, b2l_ma,
           W2r_ma, M1, bM1, M2, bM2):
    src = edge_index[0]
    dst = edge_index[1]
    n_acc = x_account.shape[0]
    n_mer = x_merchant.shape[0]
    e = src.shape[0]

    ones = jnp.ones((e,), jnp.float32)
    cnt_mer = jnp.maximum(
        jax.ops.segment_sum(ones, dst, num_segments=n_mer), 1.0)[:, None]
    cnt_acc = jnp.maximum(
        jax.ops.segment_sum(ones, src, num_segments=n_acc), 1.0)[:, None]

    m1m = _seg_mean(jnp.take(x_account, src, axis=0), dst, n_mer, cnt_mer)
    m1a = _seg_mean(jnp.take(x_merchant, dst, axis=0), src, n_acc, cnt_acc)
    mer1 = jax.nn.relu(m1m @ W1l_am + b1l_am + x_merchant @ W1r_am)
    acc1 = jax.nn.relu(m1a @ W1l_ma + b1l_ma + x_account @ W1r_ma)

    m2m = _seg_mean(jnp.take(acc1, src, axis=0), dst, n_mer, cnt_mer)
    m2a = _seg_mean(jnp.take(mer1, dst, axis=0), src, n_acc, cnt_acc)
    mer2 = jax.nn.relu(m2m @ W2l_am + b2l_am + mer1 @ W2r_am)
    acc2 = jax.nn.relu(m2a @ W2l_ma + b2l_ma + acc1 @ W2r_ma)

    a_proj = acc2 @ M1[:32] + bM1
    b_proj = mer2 @ M1[32:]
    ag = jnp.take(a_proj, src, axis=0)
    bg = jnp.take(b_proj, dst, axis=0)
    logit = _edge_score(ag, bg, M2)[:, 0] + bM2[0]
    return logit


# speedup vs baseline: 8.1589x; 8.1589x over previous
import functools

import jax
import jax.numpy as jnp
from jax import lax
from jax.experimental import pallas as pl
from jax.experimental.pallas import tpu as pltpu
from jax.experimental.pallas import tpu_sc as plsc

NC = 2
NS = 16
D16 = 16
K = 1000
ZROWS = 784
NPAD = 100352

_MESH = plsc.VectorSubcoreMesh(core_axis_name="c", subcore_axis_name="s")


def _fill_rows(ref, n_rows, value):
    val = jnp.full((16,), value, jnp.float32)

    def body(i, carry):
        ref[i, :] = val
        return carry

    lax.fori_loop(0, n_rows, body, 0)


def _zero_table(tbl, zbuf, s, rows_per_tile):
    base = s * rows_per_tile
    n = rows_per_tile // ZROWS

    def body(j, carry):
        pltpu.sync_copy(zbuf, tbl.at[pl.ds(base + j * ZROWS, ZROWS)])
        return carry

    lax.fori_loop(0, n, body, 0)


def _sc_counts(src, dst):
    e = src.shape[0]
    per_tile = e // NS
    n_chunks = per_tile // K

    def body(src_h, dst_h, out0, out1, tbl, zbuf, ones_v, sidx_v):
        c = lax.axis_index("c")
        s = lax.axis_index("s")
        rows_per_tile = out0.shape[0] // NS
        _fill_rows(zbuf, ZROWS, 0.0)
        _fill_rows(ones_v, K, 1.0)
        _zero_table(tbl, zbuf, s, rows_per_tile)
        plsc.subcore_barrier()

        def accum(idx_h):
            def chunk(j, carry):
                off = s * per_tile + j * K
                pltpu.sync_copy(idx_h.at[pl.ds(off, K)], sidx_v)
                pltpu.sync_copy(ones_v, tbl.at[sidx_v], add=True)
                return carry

            lax.fori_loop(0, n_chunks, chunk, 0)

        @pl.when(c == 0)
        def _():
            accum(src_h)

        @pl.when(c == 1)
        def _():
            accum(dst_h)

        plsc.subcore_barrier()
        base = s * rows_per_tile

        @pl.when(c == 0)
        def _():
            pltpu.sync_copy(tbl.at[pl.ds(base, rows_per_tile)],
                            out0.at[pl.ds(base, rows_per_tile)])

        @pl.when(c == 1)
        def _():
            pltpu.sync_copy(tbl.at[pl.ds(base, rows_per_tile)],
                            out1.at[pl.ds(base, rows_per_tile)])

    fn = pl.kernel(
        body,
        out_type=(jax.ShapeDtypeStruct((NPAD, D16), jnp.float32),
                  jax.ShapeDtypeStruct((NPAD, D16), jnp.float32)),
        mesh=_MESH,
        compiler_params=pltpu.CompilerParams(use_tc_tiling_on_sc=False),
        scratch_types=[
            pltpu.VMEM_SHARED((NPAD, D16), jnp.float32),
            pltpu.VMEM((ZROWS, D16), jnp.float32),
            pltpu.VMEM((K, D16), jnp.float32),
            pltpu.VMEM((K,), jnp.int32),
        ],
    )
    return fn(src, dst)


def _sc_segsum(x0, g0, s0, x1, g1, s1):
    e = g0.shape[0]
    per_tile = e // NS
    n_chunks = per_tile // K

    def body(x0_h, g0_h, s0_h, x1_h, g1_h, s1_h, out0, out1,
             tbl, zbuf, gidx_v, sidx_v, rows_v, sem):
        c = lax.axis_index("c")
        s = lax.axis_index("s")
        rows_per_tile = out0.shape[0] // NS
        _fill_rows(zbuf, ZROWS, 0.0)
        _zero_table(tbl, zbuf, s, rows_per_tile)
        plsc.subcore_barrier()

        def accum(x_h, g_h, s_h):
            def chunk(j, carry):
                off = s * per_tile + j * K
                pltpu.sync_copy(g_h.at[pl.ds(off, K)], gidx_v)
                pltpu.sync_copy(s_h.at[pl.ds(off, K)], sidx_v)
                pltpu.async_copy(x_h.at[gidx_v], rows_v, sem).wait()
                pltpu.sync_copy(rows_v, tbl.at[sidx_v], add=True)
                return carry

            lax.fori_loop(0, n_chunks, chunk, 0)

        @pl.when(c == 0)
        def _():
            accum(x0_h, g0_h, s0_h)

        @pl.when(c == 1)
        def _():
            accum(x1_h, g1_h, s1_h)

        plsc.subcore_barrier()
        base = s * rows_per_tile

        @pl.when(c == 0)
        def _():
            pltpu.sync_copy(tbl.at[pl.ds(base, rows_per_tile)],
                            out0.at[pl.ds(base, rows_per_tile)])

        @pl.when(c == 1)
        def _():
            pltpu.sync_copy(tbl.at[pl.ds(base, rows_per_tile)],
                            out1.at[pl.ds(base, rows_per_tile)])

    fn = pl.kernel(
        body,
        out_type=(jax.ShapeDtypeStruct((NPAD, D16), jnp.float32),
                  jax.ShapeDtypeStruct((NPAD, D16), jnp.float32)),
        mesh=_MESH,
        compiler_params=pltpu.CompilerParams(use_tc_tiling_on_sc=False),
        scratch_types=[
            pltpu.VMEM_SHARED((NPAD, D16), jnp.float32),
            pltpu.VMEM((ZROWS, D16), jnp.float32),
            pltpu.VMEM((K,), jnp.int32),
            pltpu.VMEM((K,), jnp.int32),
            pltpu.VMEM((K, D16), jnp.float32),
            pltpu.SemaphoreType.DMA,
        ],
    )
    return fn(x0, g0, s0, x1, g1, s1)


def _sc_edge_gather(a, b, src, dst):
    e = src.shape[0]
    d = a.shape[1]
    per_tile = e // (NC * NS)
    n_chunks = per_tile // K

    def body(a_h, b_h, src_h, dst_h, ag, bg,
             aidx_v, bidx_v, arows_v, brows_v, sem_a, sem_b):
        wid = lax.axis_index("s") * NC + lax.axis_index("c")

        def chunk(j, carry):
            off = wid * per_tile + j * K
            pltpu.sync_copy(src_h.at[pl.ds(off, K)], aidx_v)
            pltpu.sync_copy(dst_h.at[pl.ds(off, K)], bidx_v)
            cp_a = pltpu.async_copy(a_h.at[aidx_v], arows_v, sem_a)
            cp_b = pltpu.async_copy(b_h.at[bidx_v], brows_v, sem_b)
            cp_a.wait()
            cp_b.wait()
            pltpu.sync_copy(arows_v, ag.at[pl.ds(off, K)])
            pltpu.sync_copy(brows_v, bg.at[pl.ds(off, K)])
            return carry

        lax.fori_loop(0, n_chunks, chunk, 0)

    fn = pl.kernel(
        body,
        out_type=(jax.ShapeDtypeStruct((e, d), jnp.float32),
                  jax.ShapeDtypeStruct((e, d), jnp.float32)),
        mesh=_MESH,
        compiler_params=pltpu.CompilerParams(use_tc_tiling_on_sc=False),
        scratch_types=[
            pltpu.VMEM((K,), jnp.int32),
            pltpu.VMEM((K,), jnp.int32),
            pltpu.VMEM((K, d), jnp.float32),
            pltpu.VMEM((K, d), jnp.float32),
            pltpu.SemaphoreType.DMA,
            pltpu.SemaphoreType.DMA,
        ],
    )
    return fn(a, b, src, dst)


N_BLOCK = 10000
E_BLOCK = 12800


def _tc_sage_body(s_ref, cnt_ref, x_ref, wl_ref, bl_ref, wr_ref, out_ref):
    cnt = jnp.maximum(cnt_ref[:, :1], 1.0)
    mean = s_ref[...] / cnt
    out_ref[...] = jnp.maximum(
        mean @ wl_ref[...] + bl_ref[...] + x_ref[...] @ wr_ref[...], 0.0)


def _tc_sage(seg, cnt, x, wl, bl, wr):
    n, d_in = seg.shape
    d_out = wl.shape[1]
    grid = n // N_BLOCK
    return pl.pallas_call(
        _tc_sage_body,
        grid=(grid,),
        in_specs=[
            pl.BlockSpec((N_BLOCK, d_in), lambda i: (i, 0)),
            pl.BlockSpec((N_BLOCK, D16), lambda i: (i, 0)),
            pl.BlockSpec((N_BLOCK, d_in), lambda i: (i, 0)),
            pl.BlockSpec((d_in, d_out), lambda i: (0, 0)),
            pl.BlockSpec((1, d_out), lambda i: (0, 0)),
            pl.BlockSpec((d_in, d_out), lambda i: (0, 0)),
        ],
        out_specs=pl.BlockSpec((N_BLOCK, d_out), lambda i: (i, 0)),
        out_shape=jax.ShapeDtypeStruct((n, d_out), jnp.float32),
    )(seg, cnt, x, wl, bl.reshape(1, -1), wr)


def _tc_sage_proj_body(s_ref, cnt_ref, x_ref, wl_ref, bl_ref, wr_ref,
                       mp_ref, pb_ref, out_ref):
    cnt = jnp.maximum(cnt_ref[:, :1], 1.0)
    mean = s_ref[...] / cnt
    h = jnp.maximum(
        mean @ wl_ref[...] + bl_ref[...] + x_ref[...] @ wr_ref[...], 0.0)
    out_ref[...] = h @ mp_ref[...] + pb_ref[...]


def _tc_sage_proj(seg, cnt, x, wl, bl, wr, mproj, pbias):
    n, d_in = seg.shape
    d_out = wl.shape[1]
    d_proj = mproj.shape[1]
    grid = n // N_BLOCK
    return pl.pallas_call(
        _tc_sage_proj_body,
        grid=(grid,),
        in_specs=[
            pl.BlockSpec((N_BLOCK, d_in), lambda i: (i, 0)),
            pl.BlockSpec((N_BLOCK, D16), lambda i: (i, 0)),
            pl.BlockSpec((N_BLOCK, d_in), lambda i: (i, 0)),
            pl.BlockSpec((d_in, d_out), lambda i: (0, 0)),
            pl.BlockSpec((1, d_out), lambda i: (0, 0)),
            pl.BlockSpec((d_in, d_out), lambda i: (0, 0)),
            pl.BlockSpec((d_out, d_proj), lambda i: (0, 0)),
            pl.BlockSpec((1, d_proj), lambda i: (0, 0)),
        ],
        out_specs=pl.BlockSpec((N_BLOCK, d_proj), lambda i: (i, 0)),
        out_shape=jax.ShapeDtypeStruct((n, d_proj), jnp.float32),
    )(seg, cnt, x, wl, bl.reshape(1, -1), wr, mproj, pbias.reshape(1, -1))


def _edge_score_body(ag_ref, bg_ref, m2_ref, out_ref):
    h = jnp.maximum(ag_ref[...] + bg_ref[...], 0.0)
    out_ref[...] = h @ m2_ref[...]


def _edge_score(ag, bg, m2):
    e = ag.shape[0]
    grid = e // E_BLOCK
    return pl.pallas_call(
        _edge_score_body,
        grid=(grid,),
        in_specs=[
            pl.BlockSpec((E_BLOCK, 32), lambda i: (i, 0)),
            pl.BlockSpec((E_BLOCK, 32), lambda i: (i, 0)),
            pl.BlockSpec((32, 1), lambda i: (0, 0)),
        ],
        out_specs=pl.BlockSpec((E_BLOCK, 1), lambda i: (i, 0)),
        out_shape=jax.ShapeDtypeStruct((e, 1), jnp.float32),
    )(ag, bg, m2)


def kernel(x_account, x_merchant, edge_index, W1l_am, b1l_am, W1r_am,
           W1l_ma, b1l_ma, W1r_ma, W2l_am, b2l_am, W2r_am, W2l_ma, b2l_ma,
           W2r_ma, M1, bM1, M2, bM2):
    src = edge_index[0]
    dst = edge_index[1]

    n_acc = x_account.shape[0]
    n_mer = x_merchant.shape[0]

    cnt_acc, cnt_mer = _sc_counts(src, dst)
    cnt_acc = cnt_acc[:n_acc]
    cnt_mer = cnt_mer[:n_mer]

    s1m, s1a = _sc_segsum(x_account, src, dst, x_merchant, dst, src)
    mer1 = _tc_sage(s1m[:n_mer], cnt_mer, x_merchant, W1l_am, b1l_am, W1r_am)
    acc1 = _tc_sage(s1a[:n_acc], cnt_acc, x_account, W1l_ma, b1l_ma, W1r_ma)

    s2m0, s2m1 = _sc_segsum(acc1[:, :16], src, dst, acc1[:, 16:], src, dst)
    s2a0, s2a1 = _sc_segsum(mer1[:, :16], dst, src, mer1[:, 16:], dst, src)
    s2m = jnp.concatenate([s2m0[:n_mer], s2m1[:n_mer]], axis=1)
    s2a = jnp.concatenate([s2a0[:n_acc], s2a1[:n_acc]], axis=1)

    a_proj = _tc_sage_proj(s2a, cnt_acc, acc1, W2l_ma, b2l_ma, W2r_ma,
                           M1[:32], bM1)
    b_proj = _tc_sage_proj(s2m, cnt_mer, mer1, W2l_am, b2l_am, W2r_am,
                           M1[32:], jnp.zeros_like(bM1))

    ag, bg = _sc_edge_gather(a_proj, b_proj, src, dst)
    logit = _edge_score(ag, bg, M2)[:, 0] + bM2[0]
    return logit


# speedup vs baseline: 8.7244x; 1.0693x over previous
import jax
import jax.numpy as jnp
from jax import lax
from jax.experimental import pallas as pl
from jax.experimental.pallas import tpu as pltpu
from jax.experimental.pallas import tpu_sc as plsc

NC = 2
NS = 16
D16 = 16
KSEG = 500
ZROWS = 392
NPAD = 100352

_MESH = plsc.VectorSubcoreMesh(core_axis_name="c", subcore_axis_name="s")
_SC_PARAMS = pltpu.CompilerParams(use_tc_tiling_on_sc=False)


def _fill_rows(ref, n_rows, value):
    val = jnp.full((16,), value, jnp.float32)

    def body(i, carry):
        ref[i, :] = val
        return carry

    lax.fori_loop(0, n_rows, body, 0)


def _zero_table(tbl, zbuf, s, rows_per_tile):
    base = s * rows_per_tile
    n = rows_per_tile // ZROWS

    def body(j, carry):
        pltpu.sync_copy(zbuf, tbl.at[pl.ds(base + j * ZROWS, ZROWS)])
        return carry

    lax.fori_loop(0, n, body, 0)


def _write_out_tile(tbl, out, s, rows_per_tile):
    base = s * rows_per_tile
    pltpu.sync_copy(tbl.at[pl.ds(base, rows_per_tile)],
                    out.at[pl.ds(base, rows_per_tile)])


def _sc_counts(packed):
    n_rows = packed.shape[0]
    rows_per_tile = n_rows // NS
    n_outer = rows_per_tile // 2

    def body(idx_h, out0, out1, tbl, zbuf, ones_v, idx_v,
             sem_l0, sem_l1, sem_s0, sem_s1):
        c = lax.axis_index("c")
        s = lax.axis_index("s")
        out_rows = out0.shape[0] // NS
        sem_l = (sem_l0, sem_l1)
        sem_s = (sem_s0, sem_s1)
        _fill_rows(zbuf, ZROWS, 0.0)
        _fill_rows(ones_v, KSEG, 1.0)
        _zero_table(tbl, zbuf, s, out_rows)
        plsc.subcore_barrier()

        def accum(r):
            base = s * rows_per_tile

            def outer(t, carry):
                for p in range(2):
                    cj = base + 2 * t + p

                    @pl.when(t > 0)
                    def _():
                        pltpu.make_async_copy(
                            ones_v, tbl.at[idx_v.at[p, r]], sem_s[p]).wait()

                    pltpu.async_copy(idx_h.at[cj], idx_v.at[p], sem_l[p])
                for p in range(2):
                    cj = base + 2 * t + p
                    pltpu.make_async_copy(idx_h.at[cj], idx_v.at[p],
                                          sem_l[p]).wait()
                    pltpu.async_copy(ones_v, tbl.at[idx_v.at[p, r]],
                                     sem_s[p], add=True)
                return carry

            lax.fori_loop(0, n_outer, outer, 0)
            for p in range(2):
                pltpu.make_async_copy(
                    ones_v, tbl.at[idx_v.at[p, r]], sem_s[p]).wait()

        @pl.when(c == 0)
        def _():
            accum(0)

        @pl.when(c == 1)
        def _():
            accum(1)

        plsc.subcore_barrier()

        @pl.when(c == 0)
        def _():
            _write_out_tile(tbl, out0, s, out_rows)

        @pl.when(c == 1)
        def _():
            _write_out_tile(tbl, out1, s, out_rows)

    fn = pl.kernel(
        body,
        out_type=(jax.ShapeDtypeStruct((NPAD, D16), jnp.float32),
                  jax.ShapeDtypeStruct((NPAD, D16), jnp.float32)),
        mesh=_MESH,
        compiler_params=_SC_PARAMS,
        scratch_types=[
            pltpu.VMEM_SHARED((NPAD, D16), jnp.float32),
            pltpu.VMEM((ZROWS, D16), jnp.float32),
            pltpu.VMEM((KSEG, D16), jnp.float32),
            pltpu.VMEM((2, 2, KSEG), jnp.int32),
            pltpu.SemaphoreType.DMA,
            pltpu.SemaphoreType.DMA,
            pltpu.SemaphoreType.DMA,
            pltpu.SemaphoreType.DMA,
        ],
    )
    return fn(packed)


def _sc_segsum(x0, swap0, x1, swap1, packed):
    n_rows = packed.shape[0]
    rows_per_tile = n_rows // NS
    n_outer = rows_per_tile // 2

    def body(x0_h, x1_h, idx_h, out0, out1,
             tbl, zbuf, idx_v, rows_v,
             sem_l0, sem_l1, sem_g0, sem_g1, sem_s0, sem_s1):
        c = lax.axis_index("c")
        s = lax.axis_index("s")
        out_rows = out0.shape[0] // NS
        sem_l = (sem_l0, sem_l1)
        sem_g = (sem_g0, sem_g1)
        sem_s = (sem_s0, sem_s1)
        _fill_rows(zbuf, ZROWS, 0.0)
        _zero_table(tbl, zbuf, s, out_rows)
        plsc.subcore_barrier()

        def accum(x_h, swap):
            base = s * rows_per_tile
            gr, sr = (1, 0) if swap else (0, 1)

            def outer(t, carry):
                for p in range(2):
                    cj = base + 2 * t + p

                    @pl.when(t > 0)
                    def _():
                        pltpu.make_async_copy(
                            rows_v.at[p], tbl.at[idx_v.at[p, sr]],
                            sem_s[p]).wait()

                    pltpu.async_copy(idx_h.at[cj], idx_v.at[p], sem_l[p])
                for p in range(2):
                    cj = base + 2 * t + p
                    pltpu.make_async_copy(idx_h.at[cj], idx_v.at[p],
                                          sem_l[p]).wait()
                    pltpu.async_copy(x_h.at[idx_v.at[p, gr]], rows_v.at[p],
                                     sem_g[p])
                for p in range(2):
                    pltpu.make_async_copy(x_h.at[idx_v.at[p, gr]],
                                          rows_v.at[p], sem_g[p]).wait()
                    pltpu.async_copy(rows_v.at[p], tbl.at[idx_v.at[p, sr]],
                                     sem_s[p], add=True)
                return carry

            lax.fori_loop(0, n_outer, outer, 0)
            for p in range(2):
                pltpu.make_async_copy(
                    rows_v.at[p], tbl.at[idx_v.at[p, sr]], sem_s[p]).wait()

        @pl.when(c == 0)
        def _():
            accum(x0_h, swap0)

        @pl.when(c == 1)
        def _():
            accum(x1_h, swap1)

        plsc.subcore_barrier()

        @pl.when(c == 0)
        def _():
            _write_out_tile(tbl, out0, s, out_rows)

        @pl.when(c == 1)
        def _():
            _write_out_tile(tbl, out1, s, out_rows)

    fn = pl.kernel(
        body,
        out_type=(jax.ShapeDtypeStruct((NPAD, D16), jnp.float32),
                  jax.ShapeDtypeStruct((NPAD, D16), jnp.float32)),
        mesh=_MESH,
        compiler_params=_SC_PARAMS,
        scratch_types=[
            pltpu.VMEM_SHARED((NPAD, D16), jnp.float32),
            pltpu.VMEM((ZROWS, D16), jnp.float32),
            pltpu.VMEM((2, 2, KSEG), jnp.int32),
            pltpu.VMEM((2, KSEG, D16), jnp.float32),
            pltpu.SemaphoreType.DMA,
            pltpu.SemaphoreType.DMA,
            pltpu.SemaphoreType.DMA,
            pltpu.SemaphoreType.DMA,
            pltpu.SemaphoreType.DMA,
            pltpu.SemaphoreType.DMA,
        ],
    )
    return fn(x0, x1, packed)


def _sc_edge_gather(a, b, packed, e):
    d = a.shape[1]
    n_rows = packed.shape[0]
    rows_per_tile = n_rows // NS
    n_outer = rows_per_tile // 2

    def body(a_h, b_h, idx_h, ag, bg, idx_v, rows_v,
             sem_l0, sem_l1, sem_g0, sem_g1, sem_w0, sem_w1):
        c = lax.axis_index("c")
        s = lax.axis_index("s")
        sem_l = (sem_l0, sem_l1)
        sem_g = (sem_g0, sem_g1)
        sem_w = (sem_w0, sem_w1)

        def run(x_h, r, out):
            base = s * rows_per_tile

            def outer(t, carry):
                for p in range(2):
                    cj = base + 2 * t + p
                    off = cj * KSEG

                    @pl.when(t > 0)
                    def _():
                        pltpu.make_async_copy(
                            rows_v.at[p], out.at[pl.ds(off, KSEG)],
                            sem_w[p]).wait()

                    pltpu.async_copy(idx_h.at[cj], idx_v.at[p], sem_l[p])
                for p in range(2):
                    cj = base + 2 * t + p
                    pltpu.make_async_copy(idx_h.at[cj], idx_v.at[p],
                                          sem_l[p]).wait()
                    pltpu.async_copy(x_h.at[idx_v.at[p, r]], rows_v.at[p],
                                     sem_g[p])
                for p in range(2):
                    cj = base + 2 * t + p
                    off = cj * KSEG
                    pltpu.make_async_copy(x_h.at[idx_v.at[p, r]],
                                          rows_v.at[p], sem_g[p]).wait()
                    pltpu.async_copy(rows_v.at[p], out.at[pl.ds(off, KSEG)],
                                     sem_w[p])
                return carry

            lax.fori_loop(0, n_outer, outer, 0)
            for p in range(2):
                pltpu.make_async_copy(
                    rows_v.at[p], out.at[pl.ds(0, KSEG)], sem_w[p]).wait()

        @pl.when(c == 0)
        def _():
            run(a_h, 0, ag)

        @pl.when(c == 1)
        def _():
            run(b_h, 1, bg)

    fn = pl.kernel(
        body,
        out_type=(jax.ShapeDtypeStruct((e, d), jnp.float32),
                  jax.ShapeDtypeStruct((e, d), jnp.float32)),
        mesh=_MESH,
        compiler_params=_SC_PARAMS,
        scratch_types=[
            pltpu.VMEM((2, 2, KSEG), jnp.int32),
            pltpu.VMEM((2, KSEG, 32), jnp.float32),
            pltpu.SemaphoreType.DMA,
            pltpu.SemaphoreType.DMA,
            pltpu.SemaphoreType.DMA,
            pltpu.SemaphoreType.DMA,
            pltpu.SemaphoreType.DMA,
            pltpu.SemaphoreType.DMA,
        ],
    )
    return fn(a, b, packed)


N_BLOCK = 10000
E_BLOCK = 12800


def _tc_sage_body(s_ref, cnt_ref, x_ref, wl_ref, bl_ref, wr_ref, out_ref):
    cnt = jnp.maximum(cnt_ref[:, :1], 1.0)
    mean = s_ref[...] / cnt
    out_ref[...] = jnp.maximum(
        mean @ wl_ref[...] + bl_ref[...] + x_ref[...] @ wr_ref[...], 0.0)


def _tc_sage(seg, cnt, x, wl, bl, wr):
    n, d_in = seg.shape
    d_out = wl.shape[1]
    grid = n // N_BLOCK
    return pl.pallas_call(
        _tc_sage_body,
        grid=(grid,),
        in_specs=[
            pl.BlockSpec((N_BLOCK, d_in), lambda i: (i, 0)),
            pl.BlockSpec((N_BLOCK, D16), lambda i: (i, 0)),
            pl.BlockSpec((N_BLOCK, d_in), lambda i: (i, 0)),
            pl.BlockSpec((d_in, d_out), lambda i: (0, 0)),
            pl.BlockSpec((1, d_out), lambda i: (0, 0)),
            pl.BlockSpec((d_in, d_out), lambda i: (0, 0)),
        ],
        out_specs=pl.BlockSpec((N_BLOCK, d_out), lambda i: (i, 0)),
        out_shape=jax.ShapeDtypeStruct((n, d_out), jnp.float32),
    )(seg, cnt, x, wl, bl.reshape(1, -1), wr)


def _tc_sage_proj_body(s_ref, cnt_ref, x_ref, wl_ref, bl_ref, wr_ref,
                       mp_ref, pb_ref, out_ref):
    cnt = jnp.maximum(cnt_ref[:, :1], 1.0)
    mean = s_ref[...] / cnt
    h = jnp.maximum(
        mean @ wl_ref[...] + bl_ref[...] + x_ref[...] @ wr_ref[...], 0.0)
    out_ref[...] = h @ mp_ref[...] + pb_ref[...]


def _tc_sage_proj(seg, cnt, x, wl, bl, wr, mproj, pbias):
    n, d_in = seg.shape
    d_out = wl.shape[1]
    d_proj = mproj.shape[1]
    grid = n // N_BLOCK
    return pl.pallas_call(
        _tc_sage_proj_body,
        grid=(grid,),
        in_specs=[
            pl.BlockSpec((N_BLOCK, d_in), lambda i: (i, 0)),
            pl.BlockSpec((N_BLOCK, D16), lambda i: (i, 0)),
            pl.BlockSpec((N_BLOCK, d_in), lambda i: (i, 0)),
            pl.BlockSpec((d_in, d_out), lambda i: (0, 0)),
            pl.BlockSpec((1, d_out), lambda i: (0, 0)),
            pl.BlockSpec((d_in, d_out), lambda i: (0, 0)),
            pl.BlockSpec((d_out, d_proj), lambda i: (0, 0)),
            pl.BlockSpec((1, d_proj), lambda i: (0, 0)),
        ],
        out_specs=pl.BlockSpec((N_BLOCK, d_proj), lambda i: (i, 0)),
        out_shape=jax.ShapeDtypeStruct((n, d_proj), jnp.float32),
    )(seg, cnt, x, wl, bl.reshape(1, -1), wr, mproj, pbias.reshape(1, -1))


def _edge_score_body(ag_ref, bg_ref, m2_ref, out_ref):
    h = jnp.maximum(ag_ref[...] + bg_ref[...], 0.0)
    out_ref[...] = h @ m2_ref[...]


def _edge_score(ag, bg, m2):
    e = ag.shape[0]
    grid = e // E_BLOCK
    return pl.pallas_call(
        _edge_score_body,
        grid=(grid,),
        in_specs=[
            pl.BlockSpec((E_BLOCK, 32), lambda i: (i, 0)),
            pl.BlockSpec((E_BLOCK, 32), lambda i: (i, 0)),
            pl.BlockSpec((32, 1), lambda i: (0, 0)),
        ],
        out_specs=pl.BlockSpec((E_BLOCK, 1), lambda i: (i, 0)),
        out_shape=jax.ShapeDtypeStruct((e, 1), jnp.float32),
    )(ag, bg, m2)


def kernel(x_account, x_merchant, edge_index, W1l_am, b1l_am, W1r_am,
           W1l_ma, b1l_ma, W1r_ma, W2l_am, b2l_am, W2r_am, W2l_ma, b2l_ma,
           W2r_ma, M1, bM1, M2, bM2):
    n_acc = x_account.shape[0]
    n_mer = x_merchant.shape[0]
    e = edge_index.shape[1]

    packed = jnp.transpose(edge_index.reshape(2, e // KSEG, KSEG), (1, 0, 2))

    cnt_acc, cnt_mer = _sc_counts(packed)
    cnt_acc = cnt_acc[:n_acc]
    cnt_mer = cnt_mer[:n_mer]

    s1m, s1a = _sc_segsum(x_account, False, x_merchant, True, packed)
    mer1 = _tc_sage(s1m[:n_mer], cnt_mer, x_merchant, W1l_am, b1l_am, W1r_am)
    acc1 = _tc_sage(s1a[:n_acc], cnt_acc, x_account, W1l_ma, b1l_ma, W1r_ma)

    s2m0, s2m1 = _sc_segsum(acc1[:, :16], False, acc1[:, 16:], False, packed)
    s2a0, s2a1 = _sc_segsum(mer1[:, :16], True, mer1[:, 16:], True, packed)
    s2m = jnp.concatenate([s2m0[:n_mer], s2m1[:n_mer]], axis=1)
    s2a = jnp.concatenate([s2a0[:n_acc], s2a1[:n_acc]], axis=1)

    a_proj = _tc_sage_proj(s2a, cnt_acc, acc1, W2l_ma, b2l_ma, W2r_ma,
                           M1[:32], bM1)
    b_proj = _tc_sage_proj(s2m, cnt_mer, mer1, W2l_am, b2l_am, W2r_am,
                           M1[32:], jnp.zeros_like(bM1))

    ag, bg = _sc_edge_gather(a_proj, b_proj, packed, e)
    logit = _edge_score(ag, bg, M2)[:, 0] + bM2[0]
    return logit


# speedup vs baseline: 17.1672x; 1.9677x over previous
import jax
import jax.numpy as jnp
from jax import lax
from jax.experimental import pallas as pl
from jax.experimental.pallas import tpu as pltpu
from jax.experimental.pallas import tpu_sc as plsc

NC = 2
NS = 16
D16 = 16
KSEG = 500
ZROWS = 392
NPAD = 100352
NR = NPAD // 8

_MESH = plsc.VectorSubcoreMesh(core_axis_name="c", subcore_axis_name="s")
_SC_PARAMS = pltpu.CompilerParams(use_tc_tiling_on_sc=False)
_SC_PARAMS_NL = pltpu.CompilerParams(use_tc_tiling_on_sc=False,
                                     needs_layout_passes=False)


def _fill_rows(ref, n_rows, value):
    val = jnp.full((16,), value, jnp.float32)

    def body(i, carry):
        ref[i, :] = val
        return carry

    lax.fori_loop(0, n_rows, body, 0)


def _zero_table(tbl, zbuf, s, rows_per_tile):
    base = s * rows_per_tile
    n = rows_per_tile // ZROWS

    def body(j, carry):
        pltpu.sync_copy(zbuf, tbl.at[pl.ds(base + j * ZROWS, ZROWS)])
        return carry

    lax.fori_loop(0, n, body, 0)


def _write_out_tile(tbl, out, s, rows_per_tile):
    base = s * rows_per_tile
    pltpu.sync_copy(tbl.at[pl.ds(base, rows_per_tile)],
                    out.at[pl.ds(base, rows_per_tile)])


def _sc_counts(packed):
    n_rows = packed.shape[0]
    rows_per_tile = n_rows // NS
    n_outer = rows_per_tile // 2

    def body(idx_h, out0, out1, tbl, zbuf, ones_v, idx_v,
             sem_l0, sem_l1, sem_s0, sem_s1):
        c = lax.axis_index("c")
        s = lax.axis_index("s")
        out_rows = out0.shape[0] // NS
        sem_l = (sem_l0, sem_l1)
        sem_s = (sem_s0, sem_s1)
        _fill_rows(zbuf, ZROWS, 0.0)
        _fill_rows(ones_v, KSEG, 1.0)
        _zero_table(tbl, zbuf, s, out_rows)
        plsc.subcore_barrier()

        def accum(r):
            base = s * rows_per_tile

            def outer(t, carry):
                for p in range(2):
                    cj = base + 2 * t + p

                    @pl.when(t > 0)
                    def _():
                        pltpu.make_async_copy(
                            ones_v, tbl.at[idx_v.at[p, r]], sem_s[p]).wait()

                    pltpu.async_copy(idx_h.at[cj], idx_v.at[p], sem_l[p])
                for p in range(2):
                    cj = base + 2 * t + p
                    pltpu.make_async_copy(idx_h.at[cj], idx_v.at[p],
                                          sem_l[p]).wait()
                    pltpu.async_copy(ones_v, tbl.at[idx_v.at[p, r]],
                                     sem_s[p], add=True)
                return carry

            lax.fori_loop(0, n_outer, outer, 0)
            for p in range(2):
                pltpu.make_async_copy(
                    ones_v, tbl.at[idx_v.at[p, r]], sem_s[p]).wait()

        @pl.when(c == 0)
        def _():
            accum(0)

        @pl.when(c == 1)
        def _():
            accum(1)

        plsc.subcore_barrier()

        @pl.when(c == 0)
        def _():
            _write_out_tile(tbl, out0, s, out_rows)

        @pl.when(c == 1)
        def _():
            _write_out_tile(tbl, out1, s, out_rows)

    fn = pl.kernel(
        body,
        out_type=(jax.ShapeDtypeStruct((NPAD, D16), jnp.float32),
                  jax.ShapeDtypeStruct((NPAD, D16), jnp.float32)),
        mesh=_MESH,
        compiler_params=_SC_PARAMS,
        scratch_types=[
            pltpu.VMEM_SHARED((NPAD, D16), jnp.float32),
            pltpu.VMEM((ZROWS, D16), jnp.float32),
            pltpu.VMEM((KSEG, D16), jnp.float32),
            pltpu.VMEM((2, 2, KSEG), jnp.int32),
            pltpu.SemaphoreType.DMA,
            pltpu.SemaphoreType.DMA,
            pltpu.SemaphoreType.DMA,
            pltpu.SemaphoreType.DMA,
        ],
    )
    return fn(packed)


def _sc_segsum(x0, swap0, x1, swap1, packed):
    n_rows = packed.shape[0]
    rows_per_tile = n_rows // NS
    n_outer = rows_per_tile // 2

    def body(x0_h, x1_h, idx_h, out0, out1,
             tbl, zbuf, idx_v, rows_v,
             sem_l0, sem_l1, sem_g0, sem_g1, sem_s0, sem_s1):
        c = lax.axis_index("c")
        s = lax.axis_index("s")
        out_rows = out0.shape[0] // NS
        sem_l = (sem_l0, sem_l1)
        sem_g = (sem_g0, sem_g1)
        sem_s = (sem_s0, sem_s1)
        _fill_rows(zbuf, ZROWS, 0.0)
        _zero_table(tbl, zbuf, s, out_rows)
        plsc.subcore_barrier()

        def accum(x_h, swap):
            base = s * rows_per_tile
            gr, sr = (1, 0) if swap else (0, 1)

            def outer(t, carry):
                for p in range(2):
                    cj = base + 2 * t + p

                    @pl.when(t > 0)
                    def _():
                        pltpu.make_async_copy(
                            rows_v.at[p], tbl.at[idx_v.at[p, sr]],
                            sem_s[p]).wait()

                    pltpu.async_copy(idx_h.at[cj], idx_v.at[p], sem_l[p])
                for p in range(2):
                    cj = base + 2 * t + p
                    pltpu.make_async_copy(idx_h.at[cj], idx_v.at[p],
                                          sem_l[p]).wait()
                    pltpu.async_copy(x_h.at[idx_v.at[p, gr]], rows_v.at[p],
                                     sem_g[p])
                for p in range(2):
                    pltpu.make_async_copy(x_h.at[idx_v.at[p, gr]],
                                          rows_v.at[p], sem_g[p]).wait()
                    pltpu.async_copy(rows_v.at[p], tbl.at[idx_v.at[p, sr]],
                                     sem_s[p], add=True)
                return carry

            lax.fori_loop(0, n_outer, outer, 0)
            for p in range(2):
                pltpu.make_async_copy(
                    rows_v.at[p], tbl.at[idx_v.at[p, sr]], sem_s[p]).wait()

        @pl.when(c == 0)
        def _():
            accum(x0_h, swap0)

        @pl.when(c == 1)
        def _():
            accum(x1_h, swap1)

        plsc.subcore_barrier()

        @pl.when(c == 0)
        def _():
            _write_out_tile(tbl, out0, s, out_rows)

        @pl.when(c == 1)
        def _():
            _write_out_tile(tbl, out1, s, out_rows)

    fn = pl.kernel(
        body,
        out_type=(jax.ShapeDtypeStruct((NPAD, D16), jnp.float32),
                  jax.ShapeDtypeStruct((NPAD, D16), jnp.float32)),
        mesh=_MESH,
        compiler_params=_SC_PARAMS,
        scratch_types=[
            pltpu.VMEM_SHARED((NPAD, D16), jnp.float32),
            pltpu.VMEM((ZROWS, D16), jnp.float32),
            pltpu.VMEM((2, 2, KSEG), jnp.int32),
            pltpu.VMEM((2, KSEG, D16), jnp.float32),
            pltpu.SemaphoreType.DMA,
            pltpu.SemaphoreType.DMA,
            pltpu.SemaphoreType.DMA,
            pltpu.SemaphoreType.DMA,
            pltpu.SemaphoreType.DMA,
            pltpu.SemaphoreType.DMA,
        ],
    )
    return fn(x0, x1, packed)


_UNROLL = 4


def _sc_edge_score(ah0, ah1, bh0, bh1, packed, m2p):
    n_rows = packed.shape[0]
    rows_per_tile = n_rows // (NC * NS)
    n_outer = rows_per_tile // 2

    def body(a0_h, a1_h, b0_h, b1_h, idx_h, m2_h, out,
             idx_v, rows_v, sc_v, m2_v,
             sem_l0, sem_l1, sem_g0, sem_g1, sem_w0, sem_w1):
        c = lax.axis_index("c")
        s = lax.axis_index("s")
        wid = s * NC + c
        base = wid * rows_per_tile
        sem_l = (sem_l0, sem_l1)
        sem_g = (sem_g0, sem_g1)
        sem_w = (sem_w0, sem_w1)
        pltpu.sync_copy(m2_h, m2_v)
        riota = lax.iota(jnp.int32, 16)
        m2s = [m2_v[d, :] for d in range(32)]

        def gathers(p):
            pltpu.async_copy(a0_h.at[idx_v.at[p, 0]], rows_v.at[p, 0],
                             sem_g[p])
            pltpu.async_copy(a1_h.at[idx_v.at[p, 0]], rows_v.at[p, 1],
                             sem_g[p])
            pltpu.async_copy(b0_h.at[idx_v.at[p, 1]], rows_v.at[p, 2],
                             sem_g[p])
            pltpu.async_copy(b1_h.at[idx_v.at[p, 1]], rows_v.at[p, 3],
                             sem_g[p])

        def outer(t, carry):
            for p in range(2):
                cj = base + 2 * t + p

                @pl.when(t > 0)
                def _():
                    pltpu.make_async_copy(
                        sc_v.at[p], out.at[cj], sem_w[p]).wait()

                pltpu.async_copy(idx_h.at[cj], idx_v.at[p], sem_l[p])
            for p in range(2):
                cj = base + 2 * t + p
                pltpu.make_async_copy(idx_h.at[cj], idx_v.at[p],
                                      sem_l[p]).wait()
                gathers(p)
            for p in range(2):
                cj = base + 2 * t + p
                for q in range(4):
                    pltpu.make_async_copy(
                        (a0_h, a1_h, b0_h, b1_h)[q].at[idx_v.at[p, 0]],
                        rows_v.at[p, q], sem_g[p]).wait()

                zc = jnp.minimum(idx_v[p, 0, pl.ds(0, 16)], 0)

                def edge_block(g, carry2):
                    goff = jnp.minimum(g * 16, KSEG - 16)
                    rvec = riota + goff
                    acc = jnp.zeros((16,), jnp.float32)
                    for d in range(16):
                        cvec = zc + d
                        a = plsc.load_gather(rows_v.at[p, 0], [rvec, cvec])
                        bb = plsc.load_gather(rows_v.at[p, 2], [rvec, cvec])
                        acc = acc + jnp.maximum(a + bb, 0.0) * m2s[d]
                        a = plsc.load_gather(rows_v.at[p, 1], [rvec, cvec])
                        bb = plsc.load_gather(rows_v.at[p, 3], [rvec, cvec])
                        acc = acc + jnp.maximum(a + bb, 0.0) * m2s[16 + d]
                    sc_v[p, pl.ds(goff, 16)] = acc
                    return carry2

                lax.fori_loop(0, KSEG // 16 + 1, edge_block, 0)
                pltpu.async_copy(sc_v.at[p], out.at[cj], sem_w[p])
            return carry

        lax.fori_loop(0, n_outer, outer, 0)
        for p in range(2):
            pltpu.make_async_copy(sc_v.at[p], out.at[base], sem_w[p]).wait()

    fn = pl.kernel(
        body,
        out_type=jax.ShapeDtypeStruct((n_rows, KSEG), jnp.float32),
        mesh=_MESH,
        compiler_params=_SC_PARAMS_NL,
        scratch_types=[
            pltpu.VMEM((2, 2, KSEG), jnp.int32),
            pltpu.VMEM((2, 4, KSEG, D16), jnp.float32),
            pltpu.VMEM((2, KSEG), jnp.float32),
            pltpu.VMEM((32, D16), jnp.float32),
            pltpu.SemaphoreType.DMA,
            pltpu.SemaphoreType.DMA,
            pltpu.SemaphoreType.DMA,
            pltpu.SemaphoreType.DMA,
            pltpu.SemaphoreType.DMA,
            pltpu.SemaphoreType.DMA,
        ],
    )
    return fn(ah0, ah1, bh0, bh1, packed, m2p)


NB = 1568


def _bd(w):
    return jnp.kron(jnp.eye(8, dtype=w.dtype), w)


def _w16(w):
    return jnp.concatenate([_bd(w[:, :16]), _bd(w[:, 16:])], axis=1)


def _w32(w):
    return jnp.concatenate([
        jnp.concatenate([_bd(w[:16, :16]), _bd(w[:16, 16:])], axis=1),
        jnp.concatenate([_bd(w[16:, :16]), _bd(w[16:, 16:])], axis=1),
    ], axis=0)


def _b32(b):
    return jnp.concatenate([jnp.tile(b[:16], 8), jnp.tile(b[16:], 8)])


def _tc_sage16_body(s_ref, cnt_ref, x_ref, wl_ref, wr_ref, b_ref,
                    o0_ref, o1_ref):
    mean = s_ref[...] / jnp.maximum(cnt_ref[...], 1.0)
    h = jnp.maximum(
        mean @ wl_ref[...] + x_ref[...] @ wr_ref[...] + b_ref[...], 0.0)
    o0_ref[...] = h[:, :128]
    o1_ref[...] = h[:, 128:]


def _tc_sage16(seg, cnt, x, wl, bl, wr):
    blk = lambda i: (i, 0)
    return pl.pallas_call(
        _tc_sage16_body,
        grid=(NR // NB,),
        in_specs=[
            pl.BlockSpec((NB, 128), blk),
            pl.BlockSpec((NB, 128), blk),
            pl.BlockSpec((NB, 128), blk),
            pl.BlockSpec((128, 256), lambda i: (0, 0)),
            pl.BlockSpec((128, 256), lambda i: (0, 0)),
            pl.BlockSpec((1, 256), lambda i: (0, 0)),
        ],
        out_specs=[pl.BlockSpec((NB, 128), blk), pl.BlockSpec((NB, 128), blk)],
        out_shape=[jax.ShapeDtypeStruct((NR, 128), jnp.float32)] * 2,
    )(seg, cnt, x, _w16(wl), _w16(wr), _b32(bl).reshape(1, 256))


def _tc_sage32_proj_body(s0_ref, s1_ref, cnt_ref, x0_ref, x1_ref,
                         wl_ref, wr_ref, b_ref, mp_ref, pb_ref,
                         o0_ref, o1_ref):
    cnt = jnp.maximum(cnt_ref[...], 1.0)
    mean = jnp.concatenate([s0_ref[...] / cnt, s1_ref[...] / cnt], axis=1)
    x = jnp.concatenate([x0_ref[...], x1_ref[...]], axis=1)
    h = jnp.maximum(
        mean @ wl_ref[...] + x @ wr_ref[...] + b_ref[...], 0.0)
    proj = h @ mp_ref[...] + pb_ref[...]
    o0_ref[...] = proj[:, :128]
    o1_ref[...] = proj[:, 128:]


def _tc_sage32_proj(s0, s1, cnt, x0, x1, wl, bl, wr, mproj, pbias):
    blk = lambda i: (i, 0)
    w0 = lambda i: (0, 0)
    return pl.pallas_call(
        _tc_sage32_proj_body,
        grid=(NR // NB,),
        in_specs=[
            pl.BlockSpec((NB, 128), blk),
            pl.BlockSpec((NB, 128), blk),
            pl.BlockSpec((NB, 128), blk),
            pl.BlockSpec((NB, 128), blk),
            pl.BlockSpec((NB, 128), blk),
            pl.BlockSpec((256, 256), w0),
            pl.BlockSpec((256, 256), w0),
            pl.BlockSpec((1, 256), w0),
            pl.BlockSpec((256, 256), w0),
            pl.BlockSpec((1, 256), w0),
        ],
        out_specs=[pl.BlockSpec((NB, 128), blk), pl.BlockSpec((NB, 128), blk)],
        out_shape=[jax.ShapeDtypeStruct((NR, 128), jnp.float32)] * 2,
    )(s0, s1, cnt, x0, x1, _w32(wl), _w32(wr), _b32(bl).reshape(1, 256),
      _w32(mproj), pbias.reshape(1, 256))


def _pack_x(x):
    xp = x.reshape(x.shape[0] // 8, 128)
    return jnp.zeros((NR, 128), jnp.float32).at[:xp.shape[0]].set(xp)


def kernel(x_account, x_merchant, edge_index, W1l_am, b1l_am, W1r_am,
           W1l_ma, b1l_ma, W1r_ma, W2l_am, b2l_am, W2r_am, W2l_ma, b2l_ma,
           W2r_ma, M1, bM1, M2, bM2):
    e = edge_index.shape[1]

    packed = jnp.transpose(edge_index.reshape(2, e // KSEG, KSEG), (1, 0, 2))

    xa = _pack_x(x_account)
    xm = _pack_x(x_merchant)
    xa16 = xa.reshape(NPAD, D16)
    xm16 = xm.reshape(NPAD, D16)

    cnt_acc16, cnt_mer16 = _sc_counts(packed)
    cnt_acc = cnt_acc16.reshape(NR, 128)
    cnt_mer = cnt_mer16.reshape(NR, 128)

    s1m, s1a = _sc_segsum(xa16, False, xm16, True, packed)
    m1h0, m1h1 = _tc_sage16(s1m.reshape(NR, 128), cnt_mer, xm,
                            W1l_am, b1l_am, W1r_am)
    a1h0, a1h1 = _tc_sage16(s1a.reshape(NR, 128), cnt_acc, xa,
                            W1l_ma, b1l_ma, W1r_ma)

    s2m0, s2m1 = _sc_segsum(a1h0.reshape(NPAD, D16), False,
                            a1h1.reshape(NPAD, D16), False, packed)
    s2a0, s2a1 = _sc_segsum(m1h0.reshape(NPAD, D16), True,
                            m1h1.reshape(NPAD, D16), True, packed)

    ah0, ah1 = _tc_sage32_proj(s2a0.reshape(NR, 128), s2a1.reshape(NR, 128),
                               cnt_acc, a1h0, a1h1, W2l_ma, b2l_ma, W2r_ma,
                               M1[:32], _b32(bM1))
    bh0, bh1 = _tc_sage32_proj(s2m0.reshape(NR, 128), s2m1.reshape(NR, 128),
                               cnt_mer, m1h0, m1h1, W2l_am, b2l_am, W2r_am,
                               M1[32:], jnp.zeros((256,), jnp.float32))

    m2p = jnp.broadcast_to(M2[:, 0][:, None], (32, D16))
    scores = _sc_edge_score(ah0.reshape(NPAD, D16), ah1.reshape(NPAD, D16),
                            bh0.reshape(NPAD, D16), bh1.reshape(NPAD, D16),
                            packed, m2p)
    return scores.reshape(e) + bM2[0]
